# baseline (speedup 1.0000x reference)
import functools

import jax
import jax.numpy as jnp
from jax import lax
from jax.experimental import pallas as pl
from jax.experimental.pallas import tpu as pltpu
from jax.experimental.pallas import tpu_sc as plsc

N = 100000
ATOM_ID_DIM = 32
CHARGE_DIM = 8
SHAPE_ID_DIM = 16
MULT_DIM = 16
MOTIF_FEAT_SIZE = 48
MOTIF_DIM = 32
NUM_JOINS = 3
OUT_DIM = 232
NSLOT = 7

CREP = 2048
MREP = 256
SHAPE_OFS = 3 * CREP
MULT_OFS = SHAPE_OFS + 5001


CHUNK = 1000
NUM_CHUNKS = N // CHUNK


def _sc_gather(aidx, idx7, atab, smtab):
    info = plsc.get_sparse_core_info()
    nc, ns = info.num_cores, info.num_subcores
    nw = nc * ns
    chunks_per_w = -(-NUM_CHUNKS // nw)
    mesh = plsc.VectorSubcoreMesh(core_axis_name="c", subcore_axis_name="s")

    @functools.partial(
        pl.kernel,
        mesh=mesh,
        out_type=(
            jax.ShapeDtypeStruct((N, ATOM_ID_DIM), jnp.float32),
            jax.ShapeDtypeStruct((NSLOT * N, 16), jnp.float32),
        ),
        scratch_types=[
            pltpu.VMEM((CHUNK,), jnp.int32),
            pltpu.VMEM((CHUNK,), jnp.int32),
            pltpu.VMEM((CHUNK, ATOM_ID_DIM), jnp.float32),
            pltpu.VMEM((CHUNK, 16), jnp.float32),
            pltpu.SemaphoreType.DMA,
        ],
        compiler_params=pltpu.CompilerParams(use_tc_tiling_on_sc=False),
    )
    def k(aidx_hbm, idx7_hbm, atab_hbm, smtab_hbm, ga_out, gsm_out,
          aidx_v, idx_v, arows_v, grows_v, sem):
        wid = lax.axis_index("s") * nc + lax.axis_index("c")
        for c in range(chunks_per_w):
            cid = wid + nw * c

            @pl.when(cid < NUM_CHUNKS)
            def _():
                base = cid * CHUNK
                rows = pl.ds(base, CHUNK)
                pltpu.sync_copy(aidx_hbm.at[rows], aidx_v)
                pltpu.async_copy(atab_hbm.at[aidx_v], arows_v, sem).wait()
                pltpu.sync_copy(arows_v, ga_out.at[rows])
                for s in range(NSLOT):
                    srow = pl.ds(s * N + base, CHUNK)
                    pltpu.sync_copy(idx7_hbm.at[srow], idx_v)
                    pltpu.async_copy(smtab_hbm.at[idx_v], grows_v, sem).wait()
                    pltpu.sync_copy(grows_v, gsm_out.at[srow])

    return k(aidx, idx7, atab, smtab)


BR = 1000


def _tc_body(ga_ref, gsm_ref, mf_ref, pa_ref, p7_ref, pw_ref, b_ref, out_ref):
    acc = jnp.dot(ga_ref[...], pa_ref[...], preferred_element_type=jnp.float32)
    for s in range(NSLOT):
        acc += jnp.dot(gsm_ref[s], p7_ref[s], preferred_element_type=jnp.float32)
    acc += jnp.dot(mf_ref[...], pw_ref[...], preferred_element_type=jnp.float32)
    out_ref[...] = acc + b_ref[...]


def _tc_assemble(ga, gsm, mf, pa, p7, pw, b232):
    return pl.pallas_call(
        _tc_body,
        grid=(N // BR,),
        in_specs=[
            pl.BlockSpec((BR, ATOM_ID_DIM), lambda i: (i, 0)),
            pl.BlockSpec((NSLOT, BR, 16), lambda i: (0, i, 0)),
            pl.BlockSpec((BR, NUM_JOINS * MOTIF_FEAT_SIZE), lambda i: (i, 0)),
            pl.BlockSpec((ATOM_ID_DIM, OUT_DIM), lambda i: (0, 0)),
            pl.BlockSpec((NSLOT, 16, OUT_DIM), lambda i: (0, 0, 0)),
            pl.BlockSpec((NUM_JOINS * MOTIF_FEAT_SIZE, OUT_DIM), lambda i: (0, 0)),
            pl.BlockSpec((1, OUT_DIM), lambda i: (0, 0)),
        ],
        out_specs=pl.BlockSpec((BR, OUT_DIM), lambda i: (i, 0)),
        out_shape=jax.ShapeDtypeStruct((N, OUT_DIM), jnp.float32),
        compiler_params=pltpu.CompilerParams(
            dimension_semantics=("arbitrary",),
        ),
    )(ga, gsm, mf, pa, p7, pw, b232)


def kernel(atom_idx, atom_charges, motif_features, shape_classes, mult_per_atom,
           atom_id_table, atom_charge_table, shape_id_table, atom_mult_table,
           W_motif, b_motif):
    f32 = jnp.float32
    i32 = jnp.int32
    ctab16 = jnp.zeros((3, 16), f32).at[:, :CHARGE_DIM].set(atom_charge_table)
    smtab = jnp.concatenate([
        jnp.tile(ctab16, (CREP, 1)),
        shape_id_table,
        jnp.tile(atom_mult_table, (MREP, 1)),
    ], axis=0)

    rid = jnp.arange(N, dtype=i32)
    cidx = atom_charges.astype(i32) + 1 + 3 * (rid % CREP)
    sidx = (shape_classes.astype(i32) + (1 + SHAPE_OFS)).T.reshape(-1)
    midx = (mult_per_atom.astype(i32) + (1 + MULT_OFS)
            + 32 * (rid % MREP)[None, :].T).T.reshape(-1)
    idx7 = jnp.concatenate([cidx, sidx, midx])

    ga, gsm7 = _sc_gather(atom_idx.astype(i32), idx7, atom_id_table, smtab)
    gsm = gsm7.reshape(NSLOT, N, 16)

    pa = jnp.zeros((ATOM_ID_DIM, OUT_DIM), f32)
    pa = pa.at[0:32, 0:32].set(jnp.eye(32, dtype=f32))

    eye16 = jnp.eye(16, dtype=f32)
    p7 = jnp.zeros((NSLOT, 16, OUT_DIM), f32)
    p7 = p7.at[0, 0:CHARGE_DIM, 32:40].set(jnp.eye(CHARGE_DIM, dtype=f32))
    for j in range(NUM_JOINS):
        p7 = p7.at[1 + j, :, 136 + 16 * j:152 + 16 * j].set(eye16)
        p7 = p7.at[4 + j, :, 184 + 16 * j:200 + 16 * j].set(eye16)

    pw = jnp.zeros((NUM_JOINS * MOTIF_FEAT_SIZE, OUT_DIM), f32)
    for j in range(NUM_JOINS):
        pw = pw.at[j * MOTIF_FEAT_SIZE:(j + 1) * MOTIF_FEAT_SIZE,
                   40 + j * MOTIF_DIM:40 + (j + 1) * MOTIF_DIM].set(W_motif)

    b232 = jnp.zeros((1, OUT_DIM), f32)
    b232 = b232.at[0, 40:136].set(jnp.tile(b_motif, NUM_JOINS))

    return _tc_assemble(ga, gsm, motif_features, pa, p7, pw, b232)
